# Initial kernel scaffold; baseline (speedup 1.0000x reference)
import jax
import jax.numpy as jnp
from jax.experimental import pallas as pl


def kernel(ent_embs, rel_embs, time_gate_weight, time_gate_bias, W_msg, W_self, src_0, dst_0, etype_0, src_1, dst_1, etype_1, src_2, dst_2, etype_2):
    raise NotImplementedError("write your pallas kernel here")


# speedup vs baseline: 1.5576x; 1.5576x over previous
import jax
import jax.numpy as jnp
from jax import lax
from jax.experimental import pallas as pl
from jax.experimental.pallas import tpu as pltpu
from jax.experimental.pallas import tpu_sc as plsc

NUM_E = 10000
NUM_R = 230
H = 128
E = 320000
HIST = 3

NC = 2
NS = 16
NW = NC * NS
CHUNK = 128
CPW = 80
GRP = 8
NGRP = CPW // GRP
E_PAD = NW * CPW * CHUNK
N_ACC = 10112
DUMMY = NUM_E
RPS = N_ACC // NS
CR = 80

_SLOPE = (0.125 + 1.0 / 3.0) / 2.0


def _l2n(x):
    n = jnp.sqrt(jnp.sum(x * x, axis=-1, keepdims=True))
    return x / jnp.maximum(n, 1e-12)


def _sc_agg_body(ent_hbm, rel_hbm, pat_hbm, src_hbm, ety_hbm,
                 dst_hbm, dhi_hbm, dlo_hbm, ze_hbm,
                 out_e_hbm, out_c_hbm,
                 acc_e, acc_c, src_v, ety_v, dst_v, dhi_v, dlo_v,
                 buf_a, sem_g, sem_s):
    c = lax.axis_index("c")
    s = lax.axis_index("s")
    w = c * NS + s

    pltpu.sync_copy(ze_hbm, acc_e.at[pl.ds(s * RPS, RPS)])

    @pl.when(s == 0)
    def _():
        pltpu.sync_copy(ze_hbm.at[pl.ds(0, CR)], acc_c)

    plsc.subcore_barrier()

    def group(g, carry):
        base = w * CPW + g * GRP
        pltpu.sync_copy(src_hbm.at[pl.ds(base, GRP)], src_v)
        pltpu.sync_copy(ety_hbm.at[pl.ds(base, GRP)], ety_v)
        pltpu.sync_copy(dst_hbm.at[pl.ds(base, GRP)], dst_v)
        pltpu.sync_copy(dhi_hbm.at[pl.ds(base, GRP)], dhi_v)
        pltpu.sync_copy(dlo_hbm.at[pl.ds(base, GRP)], dlo_v)

        def step(j, carry2):
            drow = dst_v.at[j]
            ga = pltpu.async_copy(ent_hbm.at[src_v.at[j]], buf_a, sem_g)
            ga.wait()
            sa = pltpu.async_copy(buf_a, acc_e.at[drow], sem_s, add=True)
            sa.wait()
            gb = pltpu.async_copy(rel_hbm.at[ety_v.at[j]], buf_a, sem_g)
            gb.wait()
            sb = pltpu.async_copy(buf_a, acc_e.at[drow], sem_s, add=True)
            sb.wait()
            gc_ = pltpu.async_copy(pat_hbm.at[dlo_v.at[j]], buf_a, sem_g)
            gc_.wait()
            sc_ = pltpu.async_copy(buf_a, acc_c.at[dhi_v.at[j]], sem_s, add=True)
            sc_.wait()
            return carry2

        lax.fori_loop(0, GRP, step, 0)
        return carry

    lax.fori_loop(0, NGRP, group, 0)
    plsc.subcore_barrier()

    pltpu.sync_copy(acc_e.at[pl.ds(s * RPS, RPS)],
                    out_e_hbm.at[c].at[pl.ds(s * RPS, RPS)])

    @pl.when(s == 0)
    def _():
        pltpu.sync_copy(acc_c, out_c_hbm.at[c])


_sc_agg = pl.kernel(
    _sc_agg_body,
    out_type=(jax.ShapeDtypeStruct((NC, N_ACC, H), jnp.float32),
              jax.ShapeDtypeStruct((NC, CR, H), jnp.float32)),
    mesh=plsc.VectorSubcoreMesh(core_axis_name="c", subcore_axis_name="s",
                                num_cores=NC, num_subcores=NS),
    scratch_types=[
        pltpu.VMEM_SHARED((N_ACC, H), jnp.float32),
        pltpu.VMEM_SHARED((CR, H), jnp.float32),
        pltpu.VMEM((GRP, CHUNK), jnp.int32),
        pltpu.VMEM((GRP, CHUNK), jnp.int32),
        pltpu.VMEM((GRP, CHUNK), jnp.int32),
        pltpu.VMEM((GRP, CHUNK), jnp.int32),
        pltpu.VMEM((GRP, CHUNK), jnp.int32),
        pltpu.VMEM((CHUNK, H), jnp.float32),
        pltpu.SemaphoreType.DMA,
        pltpu.SemaphoreType.DMA,
    ],
)


def _prep_body(ent_ref, rel_ref, ent_o, rel_o):
    ent_o[...] = _l2n(ent_ref[...])
    rel_o[...] = _l2n(rel_ref[...])


_prep = pl.pallas_call(
    _prep_body,
    out_shape=(jax.ShapeDtypeStruct((N_ACC, H), jnp.float32),
               jax.ShapeDtypeStruct((2 * NUM_R, H), jnp.float32)),
)


def _update_body(acc_e_ref, acc_c_ref, ent_ref, wmsg_ref, wself_ref,
                 tgw_ref, tgb_ref, out_ref):
    acc = acc_e_ref[0] + acc_e_ref[1]
    cnt = (acc_c_ref[0] + acc_c_ref[1])[:N_ACC // H]
    acc3 = acc.reshape(N_ACC // H, H, H)
    mean = (acc3 / jnp.maximum(cnt, 1.0)[:, :, None]).reshape(N_ACC, H)
    ent = ent_ref[...]
    out = (jnp.dot(mean, wmsg_ref[...], preferred_element_type=jnp.float32)
           + jnp.dot(ent, wself_ref[...], preferred_element_type=jnp.float32))
    out = jnp.where(out >= 0, out, _SLOPE * out)
    tmp = _l2n(out)
    tw = jax.nn.sigmoid(
        jnp.dot(ent, tgw_ref[...], preferred_element_type=jnp.float32)
        + tgb_ref[...])
    out_ref[...] = _l2n(tw * tmp + (1.0 - tw) * ent)


_update = pl.pallas_call(
    _update_body,
    out_shape=jax.ShapeDtypeStruct((N_ACC, H), jnp.float32),
)


def _pad_idx(x, fill):
    x = x.astype(jnp.int32)
    pad = jnp.full((E_PAD - E,), fill, jnp.int32)
    return jnp.concatenate([x, pad]).reshape(E_PAD // CHUNK, CHUNK)


def kernel(ent_embs, rel_embs, time_gate_weight, time_gate_bias, W_msg, W_self,
           src_0, dst_0, etype_0, src_1, dst_1, etype_1, src_2, dst_2, etype_2):
    f32 = jnp.float32
    ent_pad = jnp.zeros((N_ACC, H), f32).at[:NUM_E].set(ent_embs.astype(f32))
    ent, rel_n = _prep(ent_pad, rel_embs.astype(f32))

    ze = jnp.zeros((RPS, H), f32)
    pat = jnp.eye(H, dtype=f32)
    tgb2 = time_gate_bias.astype(f32).reshape(1, H)

    outs = []
    for (s_, d_, t_) in ((src_0, dst_0, etype_0),
                         (src_1, dst_1, etype_1),
                         (src_2, dst_2, etype_2)):
        sp = _pad_idx(s_, 0)
        tp = _pad_idx(t_, 0)
        dp = _pad_idx(d_, DUMMY)
        d32 = d_.astype(jnp.int32)
        dhi = _pad_idx(lax.shift_right_logical(d32, 7), DUMMY >> 7)
        dlo = _pad_idx(lax.bitwise_and(d32, 127), DUMMY & 127)
        acc_e, acc_c = _sc_agg(ent, rel_n, pat, sp, tp, dp, dhi, dlo, ze)
        ent = _update(acc_e, acc_c, ent, W_msg.astype(f32),
                      W_self.astype(f32), time_gate_weight.astype(f32), tgb2)
        outs.append(ent[:NUM_E])

    ent_stack = jnp.stack(outs, axis=0)
    rel_stack = jnp.broadcast_to(rel_n, (HIST,) + rel_n.shape)
    return ent_stack, rel_stack


# speedup vs baseline: 2.1079x; 1.3533x over previous
import jax
import jax.numpy as jnp
from jax import lax
from jax.experimental import pallas as pl
from jax.experimental.pallas import tpu as pltpu
from jax.experimental.pallas import tpu_sc as plsc

NUM_E = 10000
NUM_R = 230
H = 128
E = 320000
HIST = 3

NC = 2
NS = 16
NW = NC * NS
CHUNK = 128
CPW = 80
GRP = 8
NGRP = CPW // GRP
E_PAD = NW * CPW * CHUNK
N_ACC = 10112
DUMMY = NUM_E
RPS = N_ACC // NS
CR = 80

_SLOPE = (0.125 + 1.0 / 3.0) / 2.0


def _l2n(x):
    n = jnp.sqrt(jnp.sum(x * x, axis=-1, keepdims=True))
    return x / jnp.maximum(n, 1e-12)


def _sc_agg_body(ent_hbm, rel_hbm, pat_hbm, src_hbm, ety_hbm,
                 dst_hbm, dhi_hbm, dlo_hbm, ze_hbm,
                 out_e_hbm, out_c_hbm,
                 acc_e, acc_c, src_v, ety_v, dst_v, dhi_v, dlo_v,
                 buf_a, buf_b, sem_g, sem_s, sem_g2, sem_s2):
    c = lax.axis_index("c")
    s = lax.axis_index("s")
    w = c * NS + s

    pltpu.sync_copy(ze_hbm, acc_e.at[pl.ds(s * RPS, RPS)])

    @pl.when(s == 0)
    def _():
        pltpu.sync_copy(ze_hbm.at[pl.ds(0, CR)], acc_c)

    plsc.subcore_barrier()

    def group(g, carry):
        base = w * CPW + g * GRP
        pltpu.sync_copy(src_hbm.at[pl.ds(base, GRP)], src_v)
        pltpu.sync_copy(ety_hbm.at[pl.ds(base, GRP)], ety_v)
        pltpu.sync_copy(dst_hbm.at[pl.ds(base, GRP)], dst_v)
        pltpu.sync_copy(dhi_hbm.at[pl.ds(base, GRP)], dhi_v)
        pltpu.sync_copy(dlo_hbm.at[pl.ds(base, GRP)], dlo_v)

        def step(j, carry2):
            drow = dst_v.at[j]
            ga = pltpu.async_copy(ent_hbm.at[src_v.at[j]], buf_a, sem_g)
            gb = pltpu.async_copy(rel_hbm.at[ety_v.at[j]], buf_b, sem_g2)
            ga.wait()
            sa = pltpu.async_copy(buf_a, acc_e.at[drow], sem_s, add=True)
            gb.wait()
            sb = pltpu.async_copy(buf_b, acc_e.at[drow], sem_s2, add=True)
            sa.wait()
            gc_ = pltpu.async_copy(pat_hbm.at[dlo_v.at[j]], buf_a, sem_g)
            gc_.wait()
            sc_ = pltpu.async_copy(buf_a, acc_c.at[dhi_v.at[j]], sem_s, add=True)
            sb.wait()
            sc_.wait()
            return carry2

        lax.fori_loop(0, GRP, step, 0)
        return carry

    lax.fori_loop(0, NGRP, group, 0)
    plsc.subcore_barrier()

    pltpu.sync_copy(acc_e.at[pl.ds(s * RPS, RPS)],
                    out_e_hbm.at[c].at[pl.ds(s * RPS, RPS)])

    @pl.when(s == 0)
    def _():
        pltpu.sync_copy(acc_c, out_c_hbm.at[c])


_sc_agg = pl.kernel(
    _sc_agg_body,
    out_type=(jax.ShapeDtypeStruct((NC, N_ACC, H), jnp.float32),
              jax.ShapeDtypeStruct((NC, CR, H), jnp.float32)),
    mesh=plsc.VectorSubcoreMesh(core_axis_name="c", subcore_axis_name="s",
                                num_cores=NC, num_subcores=NS),
    scratch_types=[
        pltpu.VMEM_SHARED((N_ACC, H), jnp.float32),
        pltpu.VMEM_SHARED((CR, H), jnp.float32),
        pltpu.VMEM((GRP, CHUNK), jnp.int32),
        pltpu.VMEM((GRP, CHUNK), jnp.int32),
        pltpu.VMEM((GRP, CHUNK), jnp.int32),
        pltpu.VMEM((GRP, CHUNK), jnp.int32),
        pltpu.VMEM((GRP, CHUNK), jnp.int32),
        pltpu.VMEM((CHUNK, H), jnp.float32),
        pltpu.VMEM((CHUNK, H), jnp.float32),
        pltpu.SemaphoreType.DMA,
        pltpu.SemaphoreType.DMA,
        pltpu.SemaphoreType.DMA,
        pltpu.SemaphoreType.DMA,
    ],
)


def _prep_body(ent_ref, rel_ref, ent_o, rel_o):
    ent_o[...] = _l2n(ent_ref[...])
    rel_o[...] = _l2n(rel_ref[...])


_prep = pl.pallas_call(
    _prep_body,
    out_shape=(jax.ShapeDtypeStruct((N_ACC, H), jnp.float32),
               jax.ShapeDtypeStruct((2 * NUM_R, H), jnp.float32)),
)


def _update_body(acc_e_ref, acc_c_ref, ent_ref, wmsg_ref, wself_ref,
                 tgw_ref, tgb_ref, out_ref):
    acc = acc_e_ref[0] + acc_e_ref[1]
    cnt = (acc_c_ref[0] + acc_c_ref[1])[:N_ACC // H]
    acc3 = acc.reshape(N_ACC // H, H, H)
    mean = (acc3 / jnp.maximum(cnt, 1.0)[:, :, None]).reshape(N_ACC, H)
    ent = ent_ref[...]
    out = (jnp.dot(mean, wmsg_ref[...], preferred_element_type=jnp.float32)
           + jnp.dot(ent, wself_ref[...], preferred_element_type=jnp.float32))
    out = jnp.where(out >= 0, out, _SLOPE * out)
    tmp = _l2n(out)
    tw = jax.nn.sigmoid(
        jnp.dot(ent, tgw_ref[...], preferred_element_type=jnp.float32)
        + tgb_ref[...])
    out_ref[...] = _l2n(tw * tmp + (1.0 - tw) * ent)


_update = pl.pallas_call(
    _update_body,
    out_shape=jax.ShapeDtypeStruct((N_ACC, H), jnp.float32),
)


def _pad_idx(x, fill):
    x = x.astype(jnp.int32)
    pad = jnp.full((E_PAD - E,), fill, jnp.int32)
    return jnp.concatenate([x, pad]).reshape(E_PAD // CHUNK, CHUNK)


def kernel(ent_embs, rel_embs, time_gate_weight, time_gate_bias, W_msg, W_self,
           src_0, dst_0, etype_0, src_1, dst_1, etype_1, src_2, dst_2, etype_2):
    f32 = jnp.float32
    ent_pad = jnp.zeros((N_ACC, H), f32).at[:NUM_E].set(ent_embs.astype(f32))
    ent, rel_n = _prep(ent_pad, rel_embs.astype(f32))

    ze = jnp.zeros((RPS, H), f32)
    pat = jnp.eye(H, dtype=f32)
    tgb2 = time_gate_bias.astype(f32).reshape(1, H)

    outs = []
    for (s_, d_, t_) in ((src_0, dst_0, etype_0),
                         (src_1, dst_1, etype_1),
                         (src_2, dst_2, etype_2)):
        sp = _pad_idx(s_, 0)
        tp = _pad_idx(t_, 0)
        dp = _pad_idx(d_, DUMMY)
        d32 = d_.astype(jnp.int32)
        dhi = _pad_idx(lax.shift_right_logical(d32, 7), DUMMY >> 7)
        dlo = _pad_idx(lax.bitwise_and(d32, 127), DUMMY & 127)
        acc_e, acc_c = _sc_agg(ent, rel_n, pat, sp, tp, dp, dhi, dlo, ze)
        ent = _update(acc_e, acc_c, ent, W_msg.astype(f32),
                      W_self.astype(f32), time_gate_weight.astype(f32), tgb2)
        outs.append(ent[:NUM_E])

    ent_stack = jnp.stack(outs, axis=0)
    rel_stack = jnp.broadcast_to(rel_n, (HIST,) + rel_n.shape)
    return ent_stack, rel_stack


# speedup vs baseline: 3.6594x; 1.7360x over previous
import jax
import jax.numpy as jnp
from jax import lax
from jax.experimental import pallas as pl
from jax.experimental.pallas import tpu as pltpu
from jax.experimental.pallas import tpu_sc as plsc

NUM_E = 10000
NUM_R = 230
H = 128
E = 320000
HIST = 3

NC = 2
NS = 16
NW = NC * NS
CHUNK = 128
CPW = 80
GRP = 8
NGRP = CPW // GRP
E_PAD = NW * CPW * CHUNK
N_ACC = 10112
DUMMY = NUM_E
RPS = N_ACC // NS
CR = 80

_SLOPE = (0.125 + 1.0 / 3.0) / 2.0


def _l2n(x):
    n = jnp.sqrt(jnp.sum(x * x, axis=-1, keepdims=True))
    return x / jnp.maximum(n, 1e-12)


def _sc_agg_body(ent_hbm, rel_hbm, pat_hbm, src_hbm, ety_hbm,
                 dst_hbm, dhi_hbm, dlo_hbm, ze_hbm,
                 out_e_hbm, out_c_hbm,
                 acc_e, acc_c, rel_s, pat_s, src_v, ety_v, dst_v, dhi_v, dlo_v,
                 buf_a, buf_b, sem_g, sem_s, sem_g2, sem_s2):
    c = lax.axis_index("c")
    s = lax.axis_index("s")
    w = c * NS + s

    pltpu.sync_copy(ze_hbm, acc_e.at[pl.ds(s * RPS, RPS)])

    @pl.when(s == 0)
    def _():
        pltpu.sync_copy(ze_hbm.at[pl.ds(0, CR)], acc_c)
        pltpu.sync_copy(rel_hbm, rel_s)
        pltpu.sync_copy(pat_hbm, pat_s)

    plsc.subcore_barrier()

    def group(g, carry):
        base = w * CPW + g * GRP
        pltpu.sync_copy(src_hbm.at[pl.ds(base, GRP)], src_v)
        pltpu.sync_copy(ety_hbm.at[pl.ds(base, GRP)], ety_v)
        pltpu.sync_copy(dst_hbm.at[pl.ds(base, GRP)], dst_v)
        pltpu.sync_copy(dhi_hbm.at[pl.ds(base, GRP)], dhi_v)
        pltpu.sync_copy(dlo_hbm.at[pl.ds(base, GRP)], dlo_v)

        def step(j, carry2):
            drow = dst_v.at[j]
            ga = pltpu.async_copy(ent_hbm.at[src_v.at[j]], buf_a, sem_g)
            gb = pltpu.async_copy(rel_s.at[ety_v.at[j]], buf_b, sem_g2)
            gb.wait()
            sb = pltpu.async_copy(buf_b, acc_e.at[drow], sem_s2, add=True)
            sb.wait()
            gc_ = pltpu.async_copy(pat_s.at[dlo_v.at[j]], buf_b, sem_g2)
            gc_.wait()
            sc_ = pltpu.async_copy(buf_b, acc_c.at[dhi_v.at[j]], sem_s2, add=True)
            ga.wait()
            sa = pltpu.async_copy(buf_a, acc_e.at[drow], sem_s, add=True)
            sa.wait()
            sc_.wait()
            return carry2

        lax.fori_loop(0, GRP, step, 0)
        return carry

    lax.fori_loop(0, NGRP, group, 0)
    plsc.subcore_barrier()

    pltpu.sync_copy(acc_e.at[pl.ds(s * RPS, RPS)],
                    out_e_hbm.at[c].at[pl.ds(s * RPS, RPS)])

    @pl.when(s == 0)
    def _():
        pltpu.sync_copy(acc_c, out_c_hbm.at[c])


_sc_agg = pl.kernel(
    _sc_agg_body,
    out_type=(jax.ShapeDtypeStruct((NC, N_ACC, H), jnp.float32),
              jax.ShapeDtypeStruct((NC, CR, H), jnp.float32)),
    mesh=plsc.VectorSubcoreMesh(core_axis_name="c", subcore_axis_name="s",
                                num_cores=NC, num_subcores=NS),
    scratch_types=[
        pltpu.VMEM_SHARED((N_ACC, H), jnp.float32),
        pltpu.VMEM_SHARED((CR, H), jnp.float32),
        pltpu.VMEM_SHARED((2 * NUM_R, H), jnp.float32),
        pltpu.VMEM_SHARED((H, H), jnp.float32),
        pltpu.VMEM((GRP, CHUNK), jnp.int32),
        pltpu.VMEM((GRP, CHUNK), jnp.int32),
        pltpu.VMEM((GRP, CHUNK), jnp.int32),
        pltpu.VMEM((GRP, CHUNK), jnp.int32),
        pltpu.VMEM((GRP, CHUNK), jnp.int32),
        pltpu.VMEM((CHUNK, H), jnp.float32),
        pltpu.VMEM((CHUNK, H), jnp.float32),
        pltpu.SemaphoreType.DMA,
        pltpu.SemaphoreType.DMA,
        pltpu.SemaphoreType.DMA,
        pltpu.SemaphoreType.DMA,
    ],
)


def _prep_body(ent_ref, rel_ref, ent_o, rel_o):
    ent_o[...] = _l2n(ent_ref[...])
    rel_o[...] = _l2n(rel_ref[...])


_prep = pl.pallas_call(
    _prep_body,
    out_shape=(jax.ShapeDtypeStruct((N_ACC, H), jnp.float32),
               jax.ShapeDtypeStruct((2 * NUM_R, H), jnp.float32)),
)


def _update_body(acc_e_ref, acc_c_ref, ent_ref, wmsg_ref, wself_ref,
                 tgw_ref, tgb_ref, out_ref):
    acc = acc_e_ref[0] + acc_e_ref[1]
    cnt = (acc_c_ref[0] + acc_c_ref[1])[:N_ACC // H]
    acc3 = acc.reshape(N_ACC // H, H, H)
    mean = (acc3 / jnp.maximum(cnt, 1.0)[:, :, None]).reshape(N_ACC, H)
    ent = ent_ref[...]
    out = (jnp.dot(mean, wmsg_ref[...], preferred_element_type=jnp.float32)
           + jnp.dot(ent, wself_ref[...], preferred_element_type=jnp.float32))
    out = jnp.where(out >= 0, out, _SLOPE * out)
    tmp = _l2n(out)
    tw = jax.nn.sigmoid(
        jnp.dot(ent, tgw_ref[...], preferred_element_type=jnp.float32)
        + tgb_ref[...])
    out_ref[...] = _l2n(tw * tmp + (1.0 - tw) * ent)


_update = pl.pallas_call(
    _update_body,
    out_shape=jax.ShapeDtypeStruct((N_ACC, H), jnp.float32),
)


def _pad_idx(x, fill):
    x = x.astype(jnp.int32)
    pad = jnp.full((E_PAD - E,), fill, jnp.int32)
    return jnp.concatenate([x, pad]).reshape(E_PAD // CHUNK, CHUNK)


def kernel(ent_embs, rel_embs, time_gate_weight, time_gate_bias, W_msg, W_self,
           src_0, dst_0, etype_0, src_1, dst_1, etype_1, src_2, dst_2, etype_2):
    f32 = jnp.float32
    ent_pad = jnp.zeros((N_ACC, H), f32).at[:NUM_E].set(ent_embs.astype(f32))
    ent, rel_n = _prep(ent_pad, rel_embs.astype(f32))

    ze = jnp.zeros((RPS, H), f32)
    pat = jnp.eye(H, dtype=f32)
    tgb2 = time_gate_bias.astype(f32).reshape(1, H)

    outs = []
    for (s_, d_, t_) in ((src_0, dst_0, etype_0),
                         (src_1, dst_1, etype_1),
                         (src_2, dst_2, etype_2)):
        sp = _pad_idx(s_, 0)
        tp = _pad_idx(t_, 0)
        dp = _pad_idx(d_, DUMMY)
        d32 = d_.astype(jnp.int32)
        dhi = _pad_idx(lax.shift_right_logical(d32, 7), DUMMY >> 7)
        dlo = _pad_idx(lax.bitwise_and(d32, 127), DUMMY & 127)
        acc_e, acc_c = _sc_agg(ent, rel_n, pat, sp, tp, dp, dhi, dlo, ze)
        ent = _update(acc_e, acc_c, ent, W_msg.astype(f32),
                      W_self.astype(f32), time_gate_weight.astype(f32), tgb2)
        outs.append(ent[:NUM_E])

    ent_stack = jnp.stack(outs, axis=0)
    rel_stack = jnp.broadcast_to(rel_n, (HIST,) + rel_n.shape)
    return ent_stack, rel_stack


# speedup vs baseline: 3.7130x; 1.0146x over previous
import jax
import jax.numpy as jnp
from jax import lax
from jax.experimental import pallas as pl
from jax.experimental.pallas import tpu as pltpu
from jax.experimental.pallas import tpu_sc as plsc

NUM_E = 10000
NUM_R = 230
H = 128
E = 320000
HIST = 3

NC = 2
NS = 16
NW = NC * NS
CHUNK = 128
CPW = 80
GRP = 8
NGRP = CPW // GRP
E_PAD = NW * CPW * CHUNK
N_ACC = 10112
DUMMY = NUM_E
RPS = N_ACC // NS
CR = 80

_SLOPE = (0.125 + 1.0 / 3.0) / 2.0


def _l2n(x):
    n = jnp.sqrt(jnp.sum(x * x, axis=-1, keepdims=True))
    return x / jnp.maximum(n, 1e-12)


def _sc_agg_body(ent_hbm, rel_hbm, pat_hbm, src_hbm, ety_hbm,
                 dst_hbm, dhi_hbm, dlo_hbm, ze_hbm,
                 out_e_hbm, out_c_hbm,
                 acc_e, acc_c, rel_s, pat_s, src_v, ety_v, dst_v, dhi_v, dlo_v,
                 buf_a, buf_b, sem_g, sem_s, sem_g2, sem_s2):
    c = lax.axis_index("c")
    s = lax.axis_index("s")
    w = c * NS + s

    pltpu.sync_copy(ze_hbm, acc_e.at[pl.ds(s * RPS, RPS)])

    @pl.when(s == 0)
    def _():
        pltpu.sync_copy(ze_hbm.at[pl.ds(0, CR)], acc_c)
        pltpu.sync_copy(rel_hbm, rel_s)
        pltpu.sync_copy(pat_hbm, pat_s)

    plsc.subcore_barrier()

    def group(g, carry):
        base = w * CPW + g * GRP
        pltpu.sync_copy(src_hbm.at[pl.ds(base, GRP)], src_v)
        pltpu.sync_copy(ety_hbm.at[pl.ds(base, GRP)], ety_v)
        pltpu.sync_copy(dst_hbm.at[pl.ds(base, GRP)], dst_v)
        pltpu.sync_copy(dhi_hbm.at[pl.ds(base, GRP)], dhi_v)
        pltpu.sync_copy(dlo_hbm.at[pl.ds(base, GRP)], dlo_v)

        def pair(i, carry2):
            j0 = 2 * i
            j1 = 2 * i + 1
            d0 = dst_v.at[j0]
            d1 = dst_v.at[j1]
            ga0 = pltpu.async_copy(ent_hbm.at[src_v.at[j0]], buf_a, sem_g)
            gb0 = pltpu.async_copy(rel_s.at[ety_v.at[j0]], buf_b, sem_g2)
            gb0.wait()
            sb0 = pltpu.async_copy(buf_b, acc_e.at[d0], sem_s2, add=True)
            sb0.wait()
            gc0 = pltpu.async_copy(pat_s.at[dlo_v.at[j0]], buf_b, sem_g2)
            gc0.wait()
            sc0 = pltpu.async_copy(buf_b, acc_c.at[dhi_v.at[j0]], sem_s2, add=True)
            ga0.wait()
            sa0 = pltpu.async_copy(buf_a, acc_e.at[d0], sem_s, add=True)
            sc0.wait()
            ga1 = pltpu.async_copy(ent_hbm.at[src_v.at[j1]], buf_b, sem_g2)
            sa0.wait()
            gb1 = pltpu.async_copy(rel_s.at[ety_v.at[j1]], buf_a, sem_g)
            gb1.wait()
            sb1 = pltpu.async_copy(buf_a, acc_e.at[d1], sem_s, add=True)
            sb1.wait()
            gc1 = pltpu.async_copy(pat_s.at[dlo_v.at[j1]], buf_a, sem_g)
            gc1.wait()
            sc1 = pltpu.async_copy(buf_a, acc_c.at[dhi_v.at[j1]], sem_s, add=True)
            ga1.wait()
            sa1 = pltpu.async_copy(buf_b, acc_e.at[d1], sem_s2, add=True)
            sc1.wait()
            sa1.wait()
            return carry2

        lax.fori_loop(0, GRP // 2, pair, 0)
        return carry

    lax.fori_loop(0, NGRP, group, 0)
    plsc.subcore_barrier()

    pltpu.sync_copy(acc_e.at[pl.ds(s * RPS, RPS)],
                    out_e_hbm.at[c].at[pl.ds(s * RPS, RPS)])

    @pl.when(s == 0)
    def _():
        pltpu.sync_copy(acc_c, out_c_hbm.at[c])


_sc_agg = pl.kernel(
    _sc_agg_body,
    out_type=(jax.ShapeDtypeStruct((NC, N_ACC, H), jnp.float32),
              jax.ShapeDtypeStruct((NC, CR, H), jnp.float32)),
    mesh=plsc.VectorSubcoreMesh(core_axis_name="c", subcore_axis_name="s",
                                num_cores=NC, num_subcores=NS),
    scratch_types=[
        pltpu.VMEM_SHARED((N_ACC, H), jnp.float32),
        pltpu.VMEM_SHARED((CR, H), jnp.float32),
        pltpu.VMEM_SHARED((2 * NUM_R, H), jnp.float32),
        pltpu.VMEM_SHARED((H, H), jnp.float32),
        pltpu.VMEM((GRP, CHUNK), jnp.int32),
        pltpu.VMEM((GRP, CHUNK), jnp.int32),
        pltpu.VMEM((GRP, CHUNK), jnp.int32),
        pltpu.VMEM((GRP, CHUNK), jnp.int32),
        pltpu.VMEM((GRP, CHUNK), jnp.int32),
        pltpu.VMEM((CHUNK, H), jnp.float32),
        pltpu.VMEM((CHUNK, H), jnp.float32),
        pltpu.SemaphoreType.DMA,
        pltpu.SemaphoreType.DMA,
        pltpu.SemaphoreType.DMA,
        pltpu.SemaphoreType.DMA,
    ],
)


def _prep_body(ent_ref, rel_ref, ent_o, rel_o):
    ent_o[...] = _l2n(ent_ref[...])
    rel_o[...] = _l2n(rel_ref[...])


_prep = pl.pallas_call(
    _prep_body,
    out_shape=(jax.ShapeDtypeStruct((N_ACC, H), jnp.float32),
               jax.ShapeDtypeStruct((2 * NUM_R, H), jnp.float32)),
)


def _update_body(acc_e_ref, acc_c_ref, ent_ref, wmsg_ref, wself_ref,
                 tgw_ref, tgb_ref, out_ref):
    acc = acc_e_ref[0] + acc_e_ref[1]
    cnt = (acc_c_ref[0] + acc_c_ref[1])[:N_ACC // H]
    acc3 = acc.reshape(N_ACC // H, H, H)
    mean = (acc3 / jnp.maximum(cnt, 1.0)[:, :, None]).reshape(N_ACC, H)
    ent = ent_ref[...]
    out = (jnp.dot(mean, wmsg_ref[...], preferred_element_type=jnp.float32)
           + jnp.dot(ent, wself_ref[...], preferred_element_type=jnp.float32))
    out = jnp.where(out >= 0, out, _SLOPE * out)
    tmp = _l2n(out)
    tw = jax.nn.sigmoid(
        jnp.dot(ent, tgw_ref[...], preferred_element_type=jnp.float32)
        + tgb_ref[...])
    out_ref[...] = _l2n(tw * tmp + (1.0 - tw) * ent)


_update = pl.pallas_call(
    _update_body,
    out_shape=jax.ShapeDtypeStruct((N_ACC, H), jnp.float32),
)


def _pad_idx(x, fill):
    x = x.astype(jnp.int32)
    pad = jnp.full((E_PAD - E,), fill, jnp.int32)
    return jnp.concatenate([x, pad]).reshape(E_PAD // CHUNK, CHUNK)


def kernel(ent_embs, rel_embs, time_gate_weight, time_gate_bias, W_msg, W_self,
           src_0, dst_0, etype_0, src_1, dst_1, etype_1, src_2, dst_2, etype_2):
    f32 = jnp.float32
    ent_pad = jnp.zeros((N_ACC, H), f32).at[:NUM_E].set(ent_embs.astype(f32))
    ent, rel_n = _prep(ent_pad, rel_embs.astype(f32))

    ze = jnp.zeros((RPS, H), f32)
    pat = jnp.eye(H, dtype=f32)
    tgb2 = time_gate_bias.astype(f32).reshape(1, H)

    outs = []
    for (s_, d_, t_) in ((src_0, dst_0, etype_0),
                         (src_1, dst_1, etype_1),
                         (src_2, dst_2, etype_2)):
        sp = _pad_idx(s_, 0)
        tp = _pad_idx(t_, 0)
        dp = _pad_idx(d_, DUMMY)
        d32 = d_.astype(jnp.int32)
        dhi = _pad_idx(lax.shift_right_logical(d32, 7), DUMMY >> 7)
        dlo = _pad_idx(lax.bitwise_and(d32, 127), DUMMY & 127)
        acc_e, acc_c = _sc_agg(ent, rel_n, pat, sp, tp, dp, dhi, dlo, ze)
        ent = _update(acc_e, acc_c, ent, W_msg.astype(f32),
                      W_self.astype(f32), time_gate_weight.astype(f32), tgb2)
        outs.append(ent[:NUM_E])

    ent_stack = jnp.stack(outs, axis=0)
    rel_stack = jnp.broadcast_to(rel_n, (HIST,) + rel_n.shape)
    return ent_stack, rel_stack
